# baseline (speedup 1.0000x reference)
import jax
import jax.numpy as jnp
from jax import lax
from jax.experimental import pallas as pl
from jax.experimental.pallas import tpu as pltpu
from jax.experimental.pallas import tpu_sc as plsc

NC = 2
NS = 16
NW = NC * NS

B = 4096
L = 50
LP = 56
EMBED = 128
BPW = B // NW
GRP = 4
NGRP = BPW // GRP
NBUF = 2


def _emb_body(idx_hbm, tab_hbm, out_hbm, idx_v, buf0, buf1, gsem, ssem):
    wid = lax.axis_index("s") * NC + lax.axis_index("c")
    bufs = (buf0, buf1)
    bat_base = wid * BPW

    pltpu.sync_copy(idx_hbm.at[wid], idx_v)

    def gstart(g, b):
        for k in range(GRP):
            pltpu.async_copy(
                tab_hbm.at[idx_v.at[g * GRP + k]], bufs[b].at[k], gsem)

    def body(g, b):
        for k in range(GRP):
            pltpu.make_async_copy(
                tab_hbm.at[idx_v.at[g * GRP + k]], bufs[b].at[k],
                gsem).wait()
        dst = out_hbm.at[pl.ds(bat_base + g * GRP, GRP)]
        pltpu.async_copy(bufs[b], dst, ssem)
        pltpu.make_async_copy(bufs[b], dst, ssem).wait()

    for b in range(NBUF):
        gstart(b, b)

    @pl.loop(0, NGRP - NBUF, step=NBUF)
    def _(g0):
        for b in range(NBUF):
            body(g0 + b, b)
            gstart(g0 + b + NBUF, b)

    for b in range(NBUF):
        body(NGRP - NBUF + b, b)


def _make_lookup():
    mesh = plsc.VectorSubcoreMesh(
        core_axis_name="c", subcore_axis_name="s",
        num_cores=NC, num_subcores=NS)
    return pl.kernel(
        _emb_body,
        out_type=jax.ShapeDtypeStruct((B, LP, EMBED), jnp.float32),
        mesh=mesh,
        scratch_types=[pltpu.VMEM((BPW, LP), jnp.int32)]
        + [pltpu.VMEM((GRP, LP, EMBED), jnp.float32) for _ in range(NBUF)]
        + [pltpu.SemaphoreType.DMA, pltpu.SemaphoreType.DMA],
    )


@jax.jit
def _emb(src_idx, tgt_idx, src_tab, tgt_tab):
    lookup = _make_lookup()
    outs = []
    for idx, tab in ((src_idx, src_tab), (tgt_idx, tgt_tab)):
        pidx = jnp.pad(idx, ((0, 0), (0, LP - L)))
        padded = lookup(pidx.reshape(NW, BPW, LP), tab)
        outs.append(lax.slice_in_dim(padded, 0, L, axis=1))
    return outs[0], outs[1]


def kernel(src_indices, tgt_indices, src_table, tgt_table):
    src_out, tgt_out = _emb(src_indices, tgt_indices, src_table, tgt_table)
    return (src_out, tgt_out)


# speedup vs baseline: 15.3631x; 15.3631x over previous
import jax
import jax.numpy as jnp
from jax import lax
from jax.experimental import pallas as pl
from jax.experimental.pallas import tpu as pltpu
from jax.experimental.pallas import tpu_sc as plsc

NC = 2
NS = 16
NW = NC * NS

B = 4096
L = 50
EMBED = 128
BPW = B // NW
NBUF = 2


def _emb_body(src_idx, tgt_idx, src_tab, tgt_tab, src_out, tgt_out,
              idx_v, buf0, buf1, gsem, ssem):
    wid = lax.axis_index("s") * NC + lax.axis_index("c")
    bufs = (buf0, buf1)
    bat_base = wid * BPW

    for idx_hbm, tab_hbm, out_hbm in (
        (src_idx, src_tab, src_out),
        (tgt_idx, tgt_tab, tgt_out),
    ):
        pltpu.sync_copy(idx_hbm.at[:, pl.ds(bat_base, BPW)], idx_v)

        def gstart(l, b):
            pltpu.async_copy(tab_hbm.at[idx_v.at[l]], bufs[b], gsem)

        def body(l, b):
            pltpu.make_async_copy(
                tab_hbm.at[idx_v.at[l]], bufs[b], gsem).wait()
            dst = out_hbm.at[l, pl.ds(bat_base, BPW)]
            pltpu.async_copy(bufs[b], dst, ssem)
            pltpu.make_async_copy(bufs[b], dst, ssem).wait()

        for b in range(NBUF):
            gstart(b, b)

        @pl.loop(0, L - NBUF, step=NBUF)
        def _(l0):
            for b in range(NBUF):
                body(l0 + b, b)
                gstart(l0 + b + NBUF, b)

        for b in range(NBUF):
            body(L - NBUF + b, b)


@jax.jit
def _emb(src_idx, tgt_idx, src_tab, tgt_tab):
    mesh = plsc.VectorSubcoreMesh(
        core_axis_name="c", subcore_axis_name="s",
        num_cores=NC, num_subcores=NS)
    f = pl.kernel(
        _emb_body,
        out_type=[
            jax.ShapeDtypeStruct((L, B, EMBED), jnp.float32),
            jax.ShapeDtypeStruct((L, B, EMBED), jnp.float32),
        ],
        mesh=mesh,
        scratch_types=[pltpu.VMEM((L, BPW), jnp.int32)]
        + [pltpu.VMEM((BPW, EMBED), jnp.float32) for _ in range(NBUF)]
        + [pltpu.SemaphoreType.DMA, pltpu.SemaphoreType.DMA],
    )
    src_out, tgt_out = f(src_idx, tgt_idx, src_tab, tgt_tab)
    return src_out, tgt_out


def kernel(src_indices, tgt_indices, src_table, tgt_table):
    src_out, tgt_out = _emb(
        src_indices.T, tgt_indices.T, src_table, tgt_table)
    return (jnp.transpose(src_out, (1, 0, 2)),
            jnp.transpose(tgt_out, (1, 0, 2)))


# speedup vs baseline: 15.6964x; 1.0217x over previous
import jax
import jax.numpy as jnp
from jax import lax
from jax.experimental import pallas as pl
from jax.experimental.pallas import tpu as pltpu
from jax.experimental.pallas import tpu_sc as plsc

NC = 2
NS = 16
NW = NC * NS

B = 4096
L = 50
EMBED = 128
BPW = B // NW
NBUF = 6
DEPTH = 3


def _emb_body(src_idx, tgt_idx, src_tab, tgt_tab, src_out, tgt_out,
              idx_v, b0, b1, b2, b3, b4, b5, gsem, ssem):
    wid = lax.axis_index("s") * NC + lax.axis_index("c")
    bufs = (b0, b1, b2, b3, b4, b5)
    bat_base = wid * BPW

    for idx_hbm, tab_hbm, out_hbm in (
        (src_idx, src_tab, src_out),
        (tgt_idx, tgt_tab, tgt_out),
    ):
        pltpu.sync_copy(idx_hbm.at[:, pl.ds(bat_base, BPW)], idx_v)

        def gstart(l, b):
            pltpu.async_copy(tab_hbm.at[idx_v.at[l]], bufs[b], gsem)

        def body(l, b, do_swait, do_gstart):
            pltpu.make_async_copy(
                tab_hbm.at[idx_v.at[l]], bufs[b], gsem).wait()
            dst = out_hbm.at[l, pl.ds(bat_base, BPW)]
            pltpu.async_copy(bufs[b], dst, ssem)
            if do_swait:
                pltpu.make_async_copy(bufs[b], dst, ssem).wait()
            if do_gstart:
                gstart(l + DEPTH, (b + DEPTH) % NBUF)

        for l in range(DEPTH):
            gstart(l, l % NBUF)

        for l in range(DEPTH):
            body(l, l % NBUF, do_swait=False, do_gstart=True)

        steady = ((L - 2 * DEPTH + 1) // NBUF) * NBUF

        @pl.loop(DEPTH, DEPTH + steady, step=NBUF)
        def _(g):
            for off in range(NBUF):
                body(g + off, (DEPTH + off) % NBUF,
                     do_swait=True, do_gstart=True)

        for l in range(DEPTH + steady, L):
            body(l, l % NBUF, do_swait=True, do_gstart=(l + DEPTH < L))

        for b in range(DEPTH):
            pltpu.make_async_copy(
                bufs[b], out_hbm.at[0, pl.ds(bat_base, BPW)], ssem).wait()


@jax.jit
def _emb(src_idx, tgt_idx, src_tab, tgt_tab):
    mesh = plsc.VectorSubcoreMesh(
        core_axis_name="c", subcore_axis_name="s",
        num_cores=NC, num_subcores=NS)
    f = pl.kernel(
        _emb_body,
        out_type=[
            jax.ShapeDtypeStruct((L, B, EMBED), jnp.float32),
            jax.ShapeDtypeStruct((L, B, EMBED), jnp.float32),
        ],
        mesh=mesh,
        scratch_types=[pltpu.VMEM((L, BPW), jnp.int32)]
        + [pltpu.VMEM((BPW, EMBED), jnp.float32) for _ in range(NBUF)]
        + [pltpu.SemaphoreType.DMA, pltpu.SemaphoreType.DMA],
        name="emb_lookup",
    )
    src_out, tgt_out = f(src_idx, tgt_idx, src_tab, tgt_tab)
    return src_out, tgt_out


def kernel(src_indices, tgt_indices, src_table, tgt_table):
    src_out, tgt_out = _emb(
        src_indices.T, tgt_indices.T, src_table, tgt_table)
    return (jnp.transpose(src_out, (1, 0, 2)),
            jnp.transpose(tgt_out, (1, 0, 2)))
